# speedup vs baseline: 1.4567x; 1.4567x over previous
import jax
import jax.numpy as jnp
from jax.experimental import pallas as pl
from jax.experimental.pallas import tpu as pltpu

E = 8
TOPK = 2
D = 1024
DFF = 704
NSH = 2
T = 2048
RSF = 2.5
EPAD = 128

TM = 1024
NTT = T // TM


def _moe_body(x_ref, gwp_ref, biasp_ref, w13_ref, w2_ref, sgu_ref, sdn_ref,
              out_ref, comb_ref, acc_ref):
    e = pl.program_id(1)
    x = x_ref[...]
    xb = x.astype(jnp.bfloat16)

    @pl.when(e == 0)
    def _router_and_shared():
        logits = jax.lax.dot_general(
            x, gwp_ref[...], (((1,), (1,)), ((), ())),
            preferred_element_type=jnp.float32)
        s = jax.nn.sigmoid(logits)
        sel = s + biasp_ref[...]
        m1 = jnp.max(sel, axis=1, keepdims=True)
        t1 = sel >= m1
        sel2 = jnp.where(t1, -jnp.inf, sel)
        m2 = jnp.max(sel2, axis=1, keepdims=True)
        t2 = sel2 >= m2
        mask = jnp.logical_or(t1, t2).astype(jnp.float32)
        sm = s * mask
        denom = jnp.sum(sm, axis=1, keepdims=True) + 1e-20
        comb_ref[...] = sm * RSF / denom

        sgu = jax.lax.dot_general(
            xb, sgu_ref[...], (((1,), (1,)), ((), ())),
            preferred_element_type=jnp.float32)
        sg = sgu[:, :DFF * NSH]
        su = sgu[:, DFF * NSH:]
        sh = (sg * jax.nn.sigmoid(sg) * su).astype(jnp.bfloat16)
        acc_ref[...] = jax.lax.dot_general(
            sh, sdn_ref[...], (((1,), (1,)), ((), ())),
            preferred_element_type=jnp.float32)

    gu = jax.lax.dot_general(
        xb, w13_ref[0], (((1,), (1,)), ((), ())),
        preferred_element_type=jnp.float32)
    g = gu[:, :DFF]
    u = gu[:, DFF:]
    h = (g * jax.nn.sigmoid(g) * u).astype(jnp.bfloat16)
    y = jax.lax.dot_general(
        h, w2_ref[0], (((1,), (1,)), ((), ())),
        preferred_element_type=jnp.float32)
    onehot = (jax.lax.broadcasted_iota(jnp.int32, (1, EPAD), 1) == e
              ).astype(jnp.float32)
    col = jnp.sum(comb_ref[...] * onehot, axis=1, keepdims=True)
    acc_ref[...] += col * y

    @pl.when(e == E - 1)
    def _finish():
        out_ref[...] = acc_ref[...]


def kernel(hidden_states, residual, gate_weight, e_score_correction_bias,
           w13, w2, shared_gate_up, shared_down):
    del residual
    gwp = jnp.zeros((EPAD, D), jnp.float32).at[:E].set(gate_weight)
    biasp = jnp.full((1, EPAD), -1e30, jnp.float32
                     ).at[0, :E].set(e_score_correction_bias)

    out = pl.pallas_call(
        _moe_body,
        grid=(NTT, E),
        in_specs=[
            pl.BlockSpec((TM, D), lambda i, e: (i, 0)),
            pl.BlockSpec((EPAD, D), lambda i, e: (0, 0)),
            pl.BlockSpec((1, EPAD), lambda i, e: (0, 0)),
            pl.BlockSpec((1, 2 * DFF, D), lambda i, e: (e, 0, 0)),
            pl.BlockSpec((1, D, DFF), lambda i, e: (e, 0, 0)),
            pl.BlockSpec((2 * DFF * NSH, D), lambda i, e: (0, 0)),
            pl.BlockSpec((D, DFF * NSH), lambda i, e: (0, 0)),
        ],
        out_specs=pl.BlockSpec((TM, D), lambda i, e: (i, 0)),
        out_shape=jax.ShapeDtypeStruct((T, D), jnp.float32),
        scratch_shapes=[
            pltpu.VMEM((TM, EPAD), jnp.float32),
            pltpu.VMEM((TM, D), jnp.float32),
        ],
        compiler_params=pltpu.CompilerParams(
            dimension_semantics=("parallel", "arbitrary"),
        ),
    )(
        hidden_states, gwp, biasp,
        w13.astype(jnp.bfloat16), w2.astype(jnp.bfloat16),
        shared_gate_up.astype(jnp.bfloat16), shared_down.astype(jnp.bfloat16),
    )
    return out
